# baseline (speedup 1.0000x reference)
import functools

import jax
import jax.numpy as jnp
from jax import lax
from jax.experimental import pallas as pl
from jax.experimental.pallas import tpu as pltpu
from jax.experimental.pallas import tpu_sc as plsc

_NC = 1
_NS = 16
_NW = _NC * _NS
_L = 16

_N = 10000 * 128
_PER_W = _N // _NW
_ACCS = 10
_CH = 8000
_NCH = _PER_W // _CH
_CH_ITERS = _CH // (_L * _ACCS)


def _sc_sumsq_body(x_hbm, out_hbm, buf0, buf1, part, sem0, sem1):
    cid = lax.axis_index("c")
    sid = lax.axis_index("s")
    wid = sid * _NC + cid
    base = wid * _PER_W
    bufs = (buf0, buf1)
    sems = (sem0, sem1)

    copies = [None, None]
    copies[0] = pltpu.async_copy(x_hbm.at[pl.ds(base, _CH)], buf0, sem0)
    zero = jnp.zeros((_L,), jnp.float32)
    accs = (zero,) * _ACCS
    for k in range(_NCH):
        if k + 1 < _NCH:
            nxt = (k + 1) % 2
            copies[nxt] = pltpu.async_copy(
                x_hbm.at[pl.ds(base + (k + 1) * _CH, _CH)], bufs[nxt], sems[nxt]
            )
        copies[k % 2].wait()
        buf = bufs[k % 2]

        def body(i, a, buf=buf):
            off = i * (_L * _ACCS)
            new = []
            for j in range(_ACCS):
                v = buf[pl.ds(off + j * _L, _L)]
                new.append(a[j] + v * v)
            return tuple(new)

        accs = lax.fori_loop(0, _CH_ITERS, body, accs)

    total = accs[0]
    for j in range(1, _ACCS):
        total = total + accs[j]
    part[...] = total
    pltpu.sync_copy(part, out_hbm.at[wid])


_sc_sumsq = functools.partial(
    pl.kernel,
    out_type=jax.ShapeDtypeStruct((_NW, _L), jnp.float32),
    mesh=plsc.VectorSubcoreMesh(
        core_axis_name="c", subcore_axis_name="s", num_cores=_NC
    ),
    scratch_types=[
        pltpu.VMEM((_CH,), jnp.float32),
        pltpu.VMEM((_CH,), jnp.float32),
        pltpu.VMEM((_L,), jnp.float32),
        pltpu.SemaphoreType.DMA,
        pltpu.SemaphoreType.DMA,
    ],
)(_sc_sumsq_body)


def _tc_finish_body(p_ref, g_ref, o_ref):
    s = jnp.sum(p_ref[...])
    o_ref[0] = jnp.sqrt(s)
    o_ref[1] = g_ref[1]
    o_ref[2] = g_ref[2]


def _tc_finish(partials, g):
    return pl.pallas_call(
        _tc_finish_body,
        out_shape=jax.ShapeDtypeStruct((3,), jnp.float32),
        in_specs=[
            pl.BlockSpec(memory_space=pltpu.VMEM),
            pl.BlockSpec(memory_space=pltpu.SMEM),
        ],
        out_specs=pl.BlockSpec(memory_space=pltpu.SMEM),
    )(partials, g)


def kernel(vertex_attr, edgeij_pair, edge_attr, g, batch):
    x = vertex_attr.reshape(-1)
    partials = _sc_sumsq(x)
    return _tc_finish(partials, g)


# speedup vs baseline: 1.1054x; 1.1054x over previous
import functools

import jax
import jax.numpy as jnp
from jax import lax
from jax.experimental import pallas as pl
from jax.experimental.pallas import tpu as pltpu
from jax.experimental.pallas import tpu_sc as plsc

_NC = 2
_NS = 16
_NW = _NC * _NS
_L = 16

_D = 128
_N_ROWS = 10000
_SC_ROWS = 2000
_TC_ROWS = _N_ROWS - _SC_ROWS
_SC_N = _SC_ROWS * _D
_PER_W = _SC_N // _NW
_ACCS = 10
_ITERS = _PER_W // (_L * _ACCS)

_TC_BLK = 1000
_TC_GRID = _TC_ROWS // _TC_BLK


def _sc_sumsq_body(x_hbm, out_hbm, buf, part):
    cid = lax.axis_index("c")
    sid = lax.axis_index("s")
    wid = sid * _NC + cid
    base = wid * _PER_W
    pltpu.sync_copy(x_hbm.at[pl.ds(base, _PER_W)], buf)

    def body(i, a):
        off = i * (_L * _ACCS)
        new = []
        for j in range(_ACCS):
            v = buf[pl.ds(off + j * _L, _L)]
            new.append(a[j] + v * v)
        return tuple(new)

    zero = jnp.zeros((_L,), jnp.float32)
    accs = lax.fori_loop(0, _ITERS, body, (zero,) * _ACCS)
    total = accs[0]
    for j in range(1, _ACCS):
        total = total + accs[j]
    part[...] = total
    pltpu.sync_copy(part, out_hbm.at[wid])


_sc_sumsq = functools.partial(
    pl.kernel,
    out_type=jax.ShapeDtypeStruct((_NW, _L), jnp.float32),
    mesh=plsc.VectorSubcoreMesh(
        core_axis_name="c", subcore_axis_name="s", num_cores=_NC
    ),
    scratch_types=[
        pltpu.VMEM((_PER_W,), jnp.float32),
        pltpu.VMEM((_L,), jnp.float32),
    ],
)(_sc_sumsq_body)


def _tc_reduce_body(x_ref, o_ref):
    i = pl.program_id(0)

    @pl.when(i == 0)
    def _():
        o_ref[0] = 0.0

    x = x_ref[...]
    o_ref[0] += jnp.sum(x * x)


_TC_ROW_OFF = _SC_ROWS // _TC_BLK


def _tc_reduce(x_full):
    return pl.pallas_call(
        _tc_reduce_body,
        grid=(_TC_GRID,),
        in_specs=[
            pl.BlockSpec((_TC_BLK, _D), lambda i: (i + _TC_ROW_OFF, 0)),
        ],
        out_specs=pl.BlockSpec(memory_space=pltpu.SMEM),
        out_shape=jax.ShapeDtypeStruct((1,), jnp.float32),
    )(x_full)


def _tc_finish_body(p_ref, t_ref, g_ref, o_ref):
    s = jnp.sum(p_ref[...]) + t_ref[0]
    o_ref[0] = jnp.sqrt(s)
    o_ref[1] = g_ref[1]
    o_ref[2] = g_ref[2]


def _tc_finish(partials, tc_sum, g):
    return pl.pallas_call(
        _tc_finish_body,
        out_shape=jax.ShapeDtypeStruct((3,), jnp.float32),
        in_specs=[
            pl.BlockSpec(memory_space=pltpu.VMEM),
            pl.BlockSpec(memory_space=pltpu.SMEM),
            pl.BlockSpec(memory_space=pltpu.SMEM),
        ],
        out_specs=pl.BlockSpec(memory_space=pltpu.SMEM),
    )(partials, tc_sum, g)


def kernel(vertex_attr, edgeij_pair, edge_attr, g, batch):
    x_flat = vertex_attr.reshape(-1)
    partials = _sc_sumsq(x_flat)
    tc_sum = _tc_reduce(vertex_attr)
    return _tc_finish(partials, tc_sum, g)


# speedup vs baseline: 1.1578x; 1.0474x over previous
import functools

import jax
import jax.numpy as jnp
from jax import lax
from jax.experimental import pallas as pl
from jax.experimental.pallas import tpu as pltpu
from jax.experimental.pallas import tpu_sc as plsc

_NC = 2
_NS = 16
_NW = _NC * _NS
_L = 16

_D = 128
_N_ROWS = 10000
_SC_ROWS = 4000
_TC_ROWS = _N_ROWS - _SC_ROWS
_SC_N = _SC_ROWS * _D
_PER_W = _SC_N // _NW
_ACCS = 10
_ITERS = _PER_W // (_L * _ACCS)

_TC_BLK = 2000
_TC_GRID = _TC_ROWS // _TC_BLK


def _sc_sumsq_body(x_hbm, out_hbm, buf, part):
    cid = lax.axis_index("c")
    sid = lax.axis_index("s")
    wid = sid * _NC + cid
    base = wid * _PER_W
    pltpu.sync_copy(x_hbm.at[pl.ds(base, _PER_W)], buf)

    def body(i, a):
        off = i * (_L * _ACCS)
        new = []
        for j in range(_ACCS):
            v = buf[pl.ds(off + j * _L, _L)]
            new.append(a[j] + v * v)
        return tuple(new)

    zero = jnp.zeros((_L,), jnp.float32)
    accs = lax.fori_loop(0, _ITERS, body, (zero,) * _ACCS)
    total = accs[0]
    for j in range(1, _ACCS):
        total = total + accs[j]
    part[...] = total
    pltpu.sync_copy(part, out_hbm.at[wid])


_sc_sumsq = functools.partial(
    pl.kernel,
    out_type=jax.ShapeDtypeStruct((_NW, _L), jnp.float32),
    mesh=plsc.VectorSubcoreMesh(
        core_axis_name="c", subcore_axis_name="s", num_cores=_NC
    ),
    scratch_types=[
        pltpu.VMEM((_PER_W,), jnp.float32),
        pltpu.VMEM((_L,), jnp.float32),
    ],
)(_sc_sumsq_body)


def _tc_reduce_body(x_ref, o_ref):
    i = pl.program_id(0)

    @pl.when(i == 0)
    def _():
        o_ref[0] = 0.0

    x = x_ref[...]
    o_ref[0] += jnp.sum(x * x)


_TC_ROW_OFF = _SC_ROWS // _TC_BLK


def _tc_reduce(x_full):
    return pl.pallas_call(
        _tc_reduce_body,
        grid=(_TC_GRID,),
        in_specs=[
            pl.BlockSpec((_TC_BLK, _D), lambda i: (i + _TC_ROW_OFF, 0)),
        ],
        out_specs=pl.BlockSpec(memory_space=pltpu.SMEM),
        out_shape=jax.ShapeDtypeStruct((1,), jnp.float32),
    )(x_full)


def _tc_finish_body(p_ref, t_ref, g_ref, o_ref):
    s = jnp.sum(p_ref[...]) + t_ref[0]
    o_ref[0] = jnp.sqrt(s)
    o_ref[1] = g_ref[1]
    o_ref[2] = g_ref[2]


def _tc_finish(partials, tc_sum, g):
    return pl.pallas_call(
        _tc_finish_body,
        out_shape=jax.ShapeDtypeStruct((3,), jnp.float32),
        in_specs=[
            pl.BlockSpec(memory_space=pltpu.VMEM),
            pl.BlockSpec(memory_space=pltpu.SMEM),
            pl.BlockSpec(memory_space=pltpu.SMEM),
        ],
        out_specs=pl.BlockSpec(memory_space=pltpu.SMEM),
    )(partials, tc_sum, g)


def kernel(vertex_attr, edgeij_pair, edge_attr, g, batch):
    x_flat = vertex_attr.reshape(-1)
    partials = _sc_sumsq(x_flat)
    tc_sum = _tc_reduce(vertex_attr)
    return _tc_finish(partials, tc_sum, g)
